# Initial kernel scaffold; baseline (speedup 1.0000x reference)
import jax
import jax.numpy as jnp
from jax.experimental import pallas as pl


def kernel(x, m, x_table, m_table, W, b, pe):
    raise NotImplementedError("write your pallas kernel here")


# speedup vs baseline: 2.6956x; 2.6956x over previous
import functools
import math

import jax
import jax.numpy as jnp
from jax import lax
from jax.experimental import pallas as pl
from jax.experimental.pallas import tpu as pltpu
from jax.experimental.pallas import tpu_sc as plsc

_CHUNK = 128


def _scale_body(xt_ref, mt_ref, xs_ref, ms_ref, *, scale):
    xs_ref[...] = xt_ref[...] * scale
    ms_ref[...] = mt_ref[...] * scale


def _scale_tables(x_table, m_table, scale):
    V, D = x_table.shape
    blk = 2000
    spec = pl.BlockSpec((blk, D), lambda i: (i, 0))
    return pl.pallas_call(
        functools.partial(_scale_body, scale=scale),
        grid=(V // blk,),
        in_specs=[spec, spec],
        out_specs=[spec, spec],
        out_shape=[jax.ShapeDtypeStruct((V, D), jnp.float32)] * 2,
    )(x_table, m_table)


def _sc_gather(table_x, table_m, xidx, midx):
    NW, n_chunks, _ = xidx.shape
    D = table_x.shape[1]
    n_rows = n_chunks * _CHUNK
    mesh = plsc.VectorSubcoreMesh(core_axis_name="c", subcore_axis_name="s")

    @functools.partial(
        pl.kernel,
        out_type=[jax.ShapeDtypeStruct((NW * n_rows, D), jnp.float32)] * 2,
        mesh=mesh,
        scratch_types=[
            pltpu.VMEM((n_chunks, _CHUNK), jnp.int32),
            pltpu.VMEM((_CHUNK, D), jnp.float32),
            pltpu.SemaphoreType.DMA,
        ],
        compiler_params=pltpu.CompilerParams(use_tc_tiling_on_sc=False),
    )
    def k(xtab, mtab, xi, mi, ox, om, idx_v, rows_v, sem):
        wid = lax.axis_index("s") * 2 + lax.axis_index("c")
        base = wid * n_rows
        for tab, idx_hbm, out_hbm in ((xtab, xi, ox), (mtab, mi, om)):
            pltpu.sync_copy(idx_hbm.at[wid], idx_v)

            @pl.loop(0, n_chunks)
            def _(j):
                pltpu.async_copy(tab.at[idx_v.at[j]], rows_v, sem).wait()
                pltpu.sync_copy(
                    rows_v, out_hbm.at[pl.ds(base + j * _CHUNK, _CHUNK)]
                )

    return k(table_x, table_m, xidx, midx)


def _gate_body(ex_ref, pe_ref, w_ref, b_ref, out_ref):
    bb, ll, d = ex_ref.shape
    xx = ex_ref[...] + pe_ref[...][None]
    x2 = xx.reshape(bb * ll, d)
    r = jnp.dot(x2, w_ref[...], preferred_element_type=jnp.float32)
    r = r + b_ref[...] + 1.0
    out_ref[...] = (x2 * r).reshape(bb, ll, d)


def _gate(emb_x, pe_l, W, b2):
    B, L, D = emb_x.shape
    bb = 16
    return pl.pallas_call(
        _gate_body,
        grid=(B // bb,),
        in_specs=[
            pl.BlockSpec((bb, L, D), lambda i: (i, 0, 0)),
            pl.BlockSpec((L, D), lambda i: (0, 0)),
            pl.BlockSpec((D, D), lambda i: (0, 0)),
            pl.BlockSpec((1, D), lambda i: (0, 0)),
        ],
        out_specs=pl.BlockSpec((bb, L, D), lambda i: (i, 0, 0)),
        out_shape=jax.ShapeDtypeStruct((B, L, D), jnp.float32),
    )(emb_x, pe_l, W, b2)


def kernel(x, m, x_table, m_table, W, b, pe):
    B, L = x.shape
    V, D = x_table.shape
    scale = math.sqrt(D)

    xt_s, mt_s = _scale_tables(x_table, m_table, scale)

    NW = 32
    total = B * L
    n_rows = total // NW
    n_chunks = n_rows // _CHUNK
    xidx = x.reshape(NW, n_chunks, _CHUNK)
    midx = m.reshape(NW, n_chunks, _CHUNK)
    ex_flat, em_flat = _sc_gather(xt_s, mt_s, xidx, midx)
    emb_x = ex_flat.reshape(B, L, D)
    emb_m = em_flat.reshape(B, L, D)

    out = _gate(emb_x, pe[0, :L, :], W, b.reshape(1, D))
    return out, emb_x, emb_m


# speedup vs baseline: 3.0819x; 1.1433x over previous
import functools
import math

import jax
import jax.numpy as jnp
from jax import lax
from jax.experimental import pallas as pl
from jax.experimental.pallas import tpu as pltpu
from jax.experimental.pallas import tpu_sc as plsc

_CHUNK = 128


def _scale_body(xt_ref, mt_ref, xs_ref, ms_ref, *, scale):
    xs_ref[...] = xt_ref[...] * scale
    ms_ref[...] = mt_ref[...] * scale


def _scale_tables(x_table, m_table, scale):
    V, D = x_table.shape
    blk = 2000
    spec = pl.BlockSpec((blk, D), lambda i: (i, 0))
    return pl.pallas_call(
        functools.partial(_scale_body, scale=scale),
        grid=(V // blk,),
        in_specs=[spec, spec],
        out_specs=[spec, spec],
        out_shape=[jax.ShapeDtypeStruct((V, D), jnp.float32)] * 2,
    )(x_table, m_table)


_S = 4


def _sc_gather(table_x, table_m, xidx, midx):
    NW, n_chunks, _ = xidx.shape
    D = table_x.shape[1]
    n_rows = n_chunks * _CHUNK
    sc_rows = _S * _CHUNK
    n_super = n_chunks // _S
    assert n_chunks % _S == 0 and n_super % 2 == 0
    mesh = plsc.VectorSubcoreMesh(core_axis_name="c", subcore_axis_name="s")

    @functools.partial(
        pl.kernel,
        out_type=[jax.ShapeDtypeStruct((NW * n_rows, D), jnp.float32)] * 2,
        mesh=mesh,
        scratch_types=[
            pltpu.VMEM((n_chunks, _CHUNK), jnp.int32),
            pltpu.VMEM((2, sc_rows, D), jnp.float32),
            pltpu.SemaphoreType.DMA,
            pltpu.SemaphoreType.DMA,
            pltpu.SemaphoreType.DMA,
            pltpu.SemaphoreType.DMA,
        ],
        compiler_params=pltpu.CompilerParams(use_tc_tiling_on_sc=False),
    )
    def k(xtab, mtab, xi, mi, ox, om, idx_v, rows2, g0, g1, o0, o1):
        wid = lax.axis_index("s") * 2 + lax.axis_index("c")
        base = wid * n_rows
        gsems = (g0, g1)
        osems = (o0, o1)
        for tab, idx_hbm, out_hbm in ((xtab, xi, ox), (mtab, mi, om)):
            pltpu.sync_copy(idx_hbm.at[wid], idx_v)

            @pl.loop(0, n_super, step=2)
            def _(gg):
                for cur in range(2):
                    g = gg + cur
                    buf = rows2.at[cur]

                    @pl.when(g >= 2)
                    def _():
                        pltpu.make_async_copy(
                            buf,
                            out_hbm.at[pl.ds(base + (g - 2) * sc_rows, sc_rows)],
                            osems[cur],
                        ).wait()

                    descs = []
                    for sblk in range(_S):
                        j = g * _S + sblk
                        descs.append(
                            pltpu.async_copy(
                                tab.at[idx_v.at[j]],
                                buf.at[pl.ds(sblk * _CHUNK, _CHUNK)],
                                gsems[cur],
                            )
                        )
                    for dsc in descs:
                        dsc.wait()
                    pltpu.async_copy(
                        buf,
                        out_hbm.at[pl.ds(base + g * sc_rows, sc_rows)],
                        osems[cur],
                    )

            for cur in range(2):
                g = n_super - 2 + cur
                pltpu.make_async_copy(
                    rows2.at[cur],
                    out_hbm.at[pl.ds(base + g * sc_rows, sc_rows)],
                    osems[cur],
                ).wait()

    return k(table_x, table_m, xidx, midx)


def _gate_body(ex_ref, pe_ref, w_ref, b_ref, out_ref):
    bb, ll, d = ex_ref.shape
    xx = ex_ref[...] + pe_ref[...][None]
    x2 = xx.reshape(bb * ll, d)
    r = jnp.dot(x2, w_ref[...], preferred_element_type=jnp.float32)
    r = r + b_ref[...] + 1.0
    out_ref[...] = (x2 * r).reshape(bb, ll, d)


def _gate(emb_x, pe_l, W, b2):
    B, L, D = emb_x.shape
    bb = 16
    return pl.pallas_call(
        _gate_body,
        grid=(B // bb,),
        in_specs=[
            pl.BlockSpec((bb, L, D), lambda i: (i, 0, 0)),
            pl.BlockSpec((L, D), lambda i: (0, 0)),
            pl.BlockSpec((D, D), lambda i: (0, 0)),
            pl.BlockSpec((1, D), lambda i: (0, 0)),
        ],
        out_specs=pl.BlockSpec((bb, L, D), lambda i: (i, 0, 0)),
        out_shape=jax.ShapeDtypeStruct((B, L, D), jnp.float32),
    )(emb_x, pe_l, W, b2)


def kernel(x, m, x_table, m_table, W, b, pe):
    B, L = x.shape
    V, D = x_table.shape
    scale = math.sqrt(D)

    xt_s, mt_s = _scale_tables(x_table, m_table, scale)

    NW = 32
    total = B * L
    n_rows = total // NW
    n_chunks = n_rows // _CHUNK
    xidx = x.reshape(NW, n_chunks, _CHUNK)
    midx = m.reshape(NW, n_chunks, _CHUNK)
    ex_flat, em_flat = _sc_gather(xt_s, mt_s, xidx, midx)
    emb_x = ex_flat.reshape(B, L, D)
    emb_m = em_flat.reshape(B, L, D)

    out = _gate(emb_x, pe[0, :L, :], W, b.reshape(1, D))
    return out, emb_x, emb_m
